# baseline (speedup 1.0000x reference)
import jax
import jax.numpy as jnp
from jax.experimental import pallas as pl
from jax.experimental.pallas import tpu as pltpu

_TEMPERATURE = 2.0
_BLOCK_R = 512


def _softmax_half(x_half, wt_ref, b_ref):
    logits = jnp.dot(x_half, wt_ref[...], preferred_element_type=jnp.float32)
    logits = (logits + b_ref[...]) * (1.0 / _TEMPERATURE)
    m = jnp.max(logits, axis=-1, keepdims=True)
    e = jnp.exp(logits - m)
    return e / jnp.sum(e, axis=-1, keepdims=True)


def _router_block(x_ref, wt_ref, b_ref, out_ref):
    d_model = wt_ref.shape[0]
    p0 = _softmax_half(x_ref[:, :d_model], wt_ref, b_ref)
    p1 = _softmax_half(x_ref[:, d_model:], wt_ref, b_ref)
    out_ref[...] = jnp.concatenate([p0, p1], axis=1)


def kernel(x, W, b):
    tokens, d_model = x.shape
    num_experts = W.shape[0]
    rows = tokens // 2
    x2 = x.reshape(rows, 2 * d_model)
    wt = W.T
    b2 = b.reshape(1, num_experts)
    br = _BLOCK_R
    out = pl.pallas_call(
        _router_block,
        grid=(rows // br,),
        in_specs=[
            pl.BlockSpec((br, 2 * d_model), lambda i: (i, 0)),
            pl.BlockSpec((d_model, num_experts), lambda i: (0, 0)),
            pl.BlockSpec((1, num_experts), lambda i: (0, 0)),
        ],
        out_specs=pl.BlockSpec((br, 2 * num_experts), lambda i: (i, 0)),
        out_shape=jax.ShapeDtypeStruct((rows, 2 * num_experts), jnp.float32),
    )(x2, wt, b2)
    return out.reshape(tokens, num_experts)


# speedup vs baseline: 4.1566x; 4.1566x over previous
import jax
import jax.numpy as jnp
from jax.experimental import pallas as pl
from jax.experimental.pallas import tpu as pltpu

_TEMPERATURE = 2.0
_BLOCK_T = 1024
_GROUP = 8
_GROUP_T = _BLOCK_T * _GROUP


def _router_block(x_ref, wt_ref, b_ref, out_hbm, obuf, osems):
    i = pl.program_id(0)
    n = pl.num_programs(0)
    g = jax.lax.div(i, _GROUP)
    j = jax.lax.rem(i, _GROUP)
    oslot = jax.lax.rem(g, 2)

    def out_copy(gg, slot):
        return pltpu.make_async_copy(
            obuf.at[slot],
            out_hbm.at[pl.ds(gg * _GROUP_T, _GROUP_T), :],
            osems.at[slot],
        )

    @pl.when(jnp.logical_and(j == 0, g >= 2))
    def _():
        out_copy(g - 2, oslot).wait()

    logits = jnp.dot(x_ref[...], wt_ref[...], preferred_element_type=jnp.float32)
    logits = (logits + b_ref[...]) * (1.0 / _TEMPERATURE)
    m = jnp.max(logits, axis=-1, keepdims=True)
    e = jnp.exp(logits - m)
    obuf[oslot, pl.ds(j * _BLOCK_T, _BLOCK_T), :] = e / jnp.sum(
        e, axis=-1, keepdims=True
    )

    @pl.when(j == _GROUP - 1)
    def _():
        out_copy(g, oslot).start()

    @pl.when(i == n - 1)
    def _():
        out_copy(g - 1, jax.lax.rem(g - 1, 2)).wait()
        out_copy(g, oslot).wait()


def kernel(x, W, b):
    tokens, d_model = x.shape
    num_experts = W.shape[0]
    wt = W.T
    b2 = b.reshape(1, num_experts)
    bt = _BLOCK_T
    return pl.pallas_call(
        _router_block,
        grid=(tokens // bt,),
        in_specs=[
            pl.BlockSpec((bt, d_model), lambda i: (i, 0)),
            pl.BlockSpec((d_model, num_experts), lambda i: (0, 0)),
            pl.BlockSpec((1, num_experts), lambda i: (0, 0)),
        ],
        out_specs=pl.BlockSpec(memory_space=pl.ANY),
        out_shape=jax.ShapeDtypeStruct((tokens, num_experts), jnp.float32),
        scratch_shapes=[
            pltpu.VMEM((2, _GROUP_T, num_experts), jnp.float32),
            pltpu.SemaphoreType.DMA((2,)),
        ],
    )(x, wt, b2)
